# baseline (speedup 1.0000x reference)
import math
import functools

import numpy as np
import jax
import jax.numpy as jnp
from jax import lax
from jax.experimental import pallas as pl
from jax.experimental.pallas import tpu as pltpu
from jax.experimental.pallas import tpu_sc as plsc

VOCAB_SIZE = 100000
D_MODEL = 1024
MAX_SEQ_LEN = 4096
SCALE = math.sqrt(D_MODEL)


def _make_pe(max_seq_len, d_model):
    position = np.arange(0, max_seq_len, dtype=np.float32)[:, None]
    div_term = np.exp(
        np.arange(0, d_model, 2, dtype=np.float32) * (-math.log(10000.0) / d_model)
    )
    pe = np.zeros((max_seq_len, d_model), dtype=np.float32)
    pe[:, 0::2] = np.sin(position * div_term)
    pe[:, 1::2] = np.cos(position * div_term)
    return pe


_PE_NP = _make_pe(MAX_SEQ_LEN, D_MODEL)

_INFO = plsc.get_sparse_core_info()
_NC, _NS, _L = _INFO.num_cores, _INFO.num_subcores, _INFO.num_lanes
_NW = _NC * _NS


def _sc_embed(table, x3, pe, B, S, D, n_chunks, C):
    n_per_w = n_chunks * C
    mesh = plsc.VectorSubcoreMesh(core_axis_name="c", subcore_axis_name="s")

    @functools.partial(
        pl.kernel,
        mesh=mesh,
        out_type=jax.ShapeDtypeStruct((B * S, D), jnp.float32),
        scratch_types=[
            pltpu.VMEM((n_chunks, C), jnp.int32),
            pltpu.VMEM((C, D), jnp.float32),
            pltpu.VMEM((C, D), jnp.float32),
            pltpu.SemaphoreType.DMA,
        ],
    )
    def k(table_hbm, x_hbm, pe_hbm, out_hbm, idx_v, rows_v, pe_v, sem):
        wid = lax.axis_index("s") * _NC + lax.axis_index("c")
        base = wid * n_per_w
        s_base = lax.rem(base, S)
        pltpu.sync_copy(x_hbm.at[wid], idx_v)

        def chunk_body(j, _):
            s0 = s_base + j * C
            pltpu.sync_copy(pe_hbm.at[pl.ds(s0, C)], pe_v)
            pltpu.async_copy(table_hbm.at[idx_v.at[j]], rows_v, sem).wait()

            def row_body(i, _):
                for kk in range(D // _L):
                    sl = pl.ds(kk * _L, _L)
                    rows_v[i, sl] = rows_v[i, sl] * SCALE + pe_v[i, sl]
                return _

            lax.fori_loop(0, C, row_body, None)
            pltpu.sync_copy(rows_v, out_hbm.at[pl.ds(base + j * C, C)])
            return _

        lax.fori_loop(0, n_chunks, chunk_body, None)

    return k(table, x3, pe)


def kernel(x, table):
    B, S = x.shape
    V, D = table.shape
    N = B * S
    n_per_w = N // _NW
    C = 32
    n_chunks = n_per_w // C
    x3 = x.astype(jnp.int32).reshape(_NW, n_chunks, C)
    pe = jnp.asarray(_PE_NP[:S])
    out = _sc_embed(table, x3, pe, B, S, D, n_chunks, C)
    return out.reshape(B, S, D)
